# baseline (speedup 1.0000x reference)
import functools

import jax
import jax.numpy as jnp
from jax import lax
from jax.experimental import pallas as pl
from jax.experimental.pallas import tpu as pltpu
from jax.experimental.pallas import tpu_sc as plsc

VOCAB = 1000000
EMBED_DIM = 32
BATCH = 4096
HIST = 50

_B = BATCH * HIST

_info = plsc.get_sparse_core_info()
_NC = _info.num_cores
_NS = _info.num_subcores
_NW = _NC * _NS
_B_PER_W = _B // _NW
_CHUNK = 1600
_NCHUNK = _B_PER_W // _CHUNK

_mesh = plsc.VectorSubcoreMesh(core_axis_name="c", subcore_axis_name="s")


@functools.partial(
    pl.kernel,
    mesh=_mesh,
    out_type=jax.ShapeDtypeStruct((_B, EMBED_DIM), jnp.float32),
    scratch_types=[
        pltpu.VMEM((_CHUNK,), jnp.int32),
        pltpu.VMEM((_CHUNK, EMBED_DIM), jnp.float32),
        pltpu.SemaphoreType.DMA,
    ],
    compiler_params=pltpu.CompilerParams(use_tc_tiling_on_sc=False),
)
def _sc_gather(table_hbm, idx_hbm, out_hbm, idx_v, rows_v, sem):
    wid = lax.axis_index("s") * _NC + lax.axis_index("c")
    base = wid * _B_PER_W
    for i in range(_NCHUNK):
        off = base + i * _CHUNK
        pltpu.sync_copy(idx_hbm.at[pl.ds(off, _CHUNK)], idx_v)
        pltpu.async_copy(table_hbm.at[idx_v], rows_v, sem).wait()
        pltpu.sync_copy(rows_v, out_hbm.at[pl.ds(off, _CHUNK)])


def kernel(inputs, embedding):
    ids = inputs.reshape(_B).astype(jnp.int32)
    out = _sc_gather(embedding, ids)
    return out.reshape(BATCH, HIST, EMBED_DIM)


# speedup vs baseline: 1.0030x; 1.0030x over previous
import functools

import jax
import jax.numpy as jnp
from jax import lax
from jax.experimental import pallas as pl
from jax.experimental.pallas import tpu as pltpu
from jax.experimental.pallas import tpu_sc as plsc

VOCAB = 1000000
EMBED_DIM = 32
BATCH = 4096
HIST = 50

_B = BATCH * HIST

_info = plsc.get_sparse_core_info()
_NC = _info.num_cores
_NS = _info.num_subcores
_NW = _NC * _NS
_B_PER_W = _B // _NW
_CHUNK = 1600
_NCHUNK = _B_PER_W // _CHUNK

_mesh = plsc.VectorSubcoreMesh(core_axis_name="c", subcore_axis_name="s")


@functools.partial(
    pl.kernel,
    mesh=_mesh,
    out_type=jax.ShapeDtypeStruct((_B, EMBED_DIM), jnp.float32),
    scratch_types=[
        pltpu.VMEM((_B_PER_W,), jnp.int32),
        pltpu.VMEM((2, _CHUNK, EMBED_DIM), jnp.float32),
        pltpu.SemaphoreType.DMA,
        pltpu.SemaphoreType.DMA,
        pltpu.SemaphoreType.DMA,
        pltpu.SemaphoreType.DMA,
    ],
    compiler_params=pltpu.CompilerParams(use_tc_tiling_on_sc=False),
)
def _sc_gather(table_hbm, idx_hbm, out_hbm, idx_v, rows_v, g0, g1, w0, w1):
    wid = lax.axis_index("s") * _NC + lax.axis_index("c")
    base = wid * _B_PER_W
    gsem = (g0, g1)
    wsem = (w0, w1)
    pltpu.sync_copy(idx_hbm.at[pl.ds(base, _B_PER_W)], idx_v)

    def gather_args(i):
        return (
            table_hbm.at[idx_v.at[pl.ds(i * _CHUNK, _CHUNK)]],
            rows_v.at[i % 2],
            gsem[i % 2],
        )

    def writeback_args(i):
        return (
            rows_v.at[i % 2],
            out_hbm.at[pl.ds(base + i * _CHUNK, _CHUNK)],
            wsem[i % 2],
        )

    pltpu.async_copy(*gather_args(0))
    for i in range(_NCHUNK):
        pltpu.make_async_copy(*gather_args(i)).wait()
        if i + 1 < _NCHUNK:
            if i >= 1:
                pltpu.make_async_copy(*writeback_args(i - 1)).wait()
            pltpu.async_copy(*gather_args(i + 1))
        pltpu.async_copy(*writeback_args(i))
    pltpu.make_async_copy(*writeback_args(_NCHUNK - 2)).wait()
    pltpu.make_async_copy(*writeback_args(_NCHUNK - 1)).wait()


def kernel(inputs, embedding):
    ids = inputs.reshape(_B).astype(jnp.int32)
    out = _sc_gather(embedding, ids)
    return out.reshape(BATCH, HIST, EMBED_DIM)


# speedup vs baseline: 1.2406x; 1.2369x over previous
import functools

import jax
import jax.numpy as jnp
from jax import lax
from jax.experimental import pallas as pl
from jax.experimental.pallas import tpu as pltpu
from jax.experimental.pallas import tpu_sc as plsc

VOCAB = 1000000
EMBED_DIM = 32
BATCH = 4096
HIST = 50

_B = BATCH * HIST

_info = plsc.get_sparse_core_info()
_NC = _info.num_cores
_NS = _info.num_subcores
_NW = _NC * _NS
_B_PER_W = _B // _NW
_NBLK = _B_PER_W // 128
_NB = 5
_NCH = _NBLK // _NB
_CH_IDS = _NB * 128
_TWORDS = _NB * 4 * 8 * 128

_mesh = plsc.VectorSubcoreMesh(core_axis_name="c", subcore_axis_name="s")


@functools.partial(
    pl.kernel,
    mesh=_mesh,
    out_type=jax.ShapeDtypeStruct((HIST, 4, 32, 1024), jnp.float32),
    scratch_types=[
        pltpu.VMEM((_B_PER_W,), jnp.int32),
        pltpu.VMEM((2, _CH_IDS, EMBED_DIM), jnp.float32),
        pltpu.VMEM((2, _TWORDS), jnp.float32),
        pltpu.SemaphoreType.DMA,
        pltpu.SemaphoreType.DMA,
        pltpu.SemaphoreType.DMA,
        pltpu.SemaphoreType.DMA,
    ],
    compiler_params=pltpu.CompilerParams(
        use_tc_tiling_on_sc=False, needs_layout_passes=False
    ),
)
def _sc_gather(table_hbm, idx_hbm, out_hbm, idx_v, g_buf, t_buf, g0, g1, w0, w1):
    wid = lax.axis_index("s") * _NC + lax.axis_index("c")
    base = wid * _B_PER_W
    gsem = (g0, g1)
    wsem = (w0, w1)
    pltpu.sync_copy(idx_hbm.at[pl.ds(base, _B_PER_W)], idx_v)

    def gather_args(ch):
        return (
            table_hbm.at[idx_v.at[pl.ds(ch * _CH_IDS, _CH_IDS)]],
            g_buf.at[ch % 2],
            gsem[ch % 2],
        )

    def wb_args(ch, lb, c4):
        blk = wid * _NBLK + ch * _NB + lb
        h = lax.shift_right_logical(blk, 5)
        bt = lax.bitwise_and(blk, 31)
        return (
            t_buf.at[ch % 2, pl.ds(c4 * (_NB * 1024) + lb * 1024, 1024)],
            out_hbm.at[h, c4, bt],
            wsem[ch % 2],
        )

    lane = lax.iota(jnp.int32, 16)
    pat = (lane // 8) * (_NB * 1024) + (lane % 8) * 128

    def transpose(ch):
        gb = g_buf.at[ch % 2]
        tb = t_buf.at[ch % 2]

        def body(m, carry):
            lb = lax.shift_right_logical(m, 7)
            j = lax.bitwise_and(m, 127)
            dst = pat + lb * 1024 + j
            for half in range(2):
                val = gb[m, pl.ds(half * 16, 16)]
                plsc.store_scatter(tb, [dst + half * (2 * _NB * 1024)], val)
            return carry

        lax.fori_loop(0, _CH_IDS, body, 0, unroll=4)

    def issue_wb(ch):
        for lb in range(_NB):
            for c4 in range(4):
                pltpu.async_copy(*wb_args(ch, lb, c4))

    def drain_wb(ch):
        for lb in range(_NB):
            for c4 in range(4):
                pltpu.make_async_copy(*wb_args(ch, lb, c4)).wait()

    pltpu.async_copy(*gather_args(0))
    for ch in range(_NCH):
        pltpu.make_async_copy(*gather_args(ch)).wait()
        if ch >= 1:
            issue_wb(ch - 1)
        if ch + 1 < _NCH:
            pltpu.async_copy(*gather_args(ch + 1))
        if ch >= 2:
            drain_wb(ch - 2)
        transpose(ch)
    drain_wb(_NCH - 2)
    issue_wb(_NCH - 1)
    drain_wb(_NCH - 1)


def kernel(inputs, embedding):
    ids = jnp.transpose(inputs).reshape(_B).astype(jnp.int32)
    v = _sc_gather(embedding, ids)
    return (
        v.reshape(HIST, 4, 32, 8, 128)
        .transpose(2, 4, 0, 1, 3)
        .reshape(BATCH, HIST, EMBED_DIM)
    )
